# Initial kernel scaffold; baseline (speedup 1.0000x reference)
import jax
import jax.numpy as jnp
from jax.experimental import pallas as pl


def kernel(z_e, codebook):
    raise NotImplementedError("write your pallas kernel here")


# speedup vs baseline: 1.1807x; 1.1807x over previous
import functools

import jax
import jax.numpy as jnp
from jax import lax
from jax.experimental import pallas as pl
from jax.experimental.pallas import tpu as pltpu
from jax.experimental.pallas import tpu_sc as plsc

_B = 16384
_K = 8192
_D = 32
_TB = 128
_HK = _K // 2


def _dist_argmin_body(z2_ref, e2_ref, zb_ref, cbtb_ref, codes_ref):
    z2 = z2_ref[...]
    e2 = e2_ref[...]
    zb = zb_ref[...]
    cbtb = cbtb_ref[...]
    ze = lax.dot_general(zb, cbtb, (((1,), (0,)), ((), ())),
                         preferred_element_type=jnp.float32)
    dist = (z2 + e2) - 2.0 * ze

    h0 = dist[:, :_HK]
    h1 = dist[:, _HK:]
    i0 = lax.broadcasted_iota(jnp.int32, (_TB, _HK), 1)
    m0 = jnp.min(h0, axis=1, keepdims=True)
    k0 = jnp.min(jnp.where(h0 == m0, i0, _K), axis=1)
    m1 = jnp.min(h1, axis=1, keepdims=True)
    k1 = jnp.min(jnp.where(h1 == m1, i0 + _HK, _K), axis=1)
    a = m0[:, 0].astype(jnp.bfloat16).astype(jnp.float32)
    codes_ref[...] = jnp.where(a <= m1[:, 0], k0, k1)


def _compute_codes(z2, e2, zb, cbtb):
    return pl.pallas_call(
        _dist_argmin_body,
        grid=(_B // _TB,),
        in_specs=[
            pl.BlockSpec((_TB, 1), lambda i: (i, 0)),
            pl.BlockSpec((1, _K), lambda i: (0, 0)),
            pl.BlockSpec((_TB, _D), lambda i: (i, 0)),
            pl.BlockSpec((_D, _K), lambda i: (0, 0)),
        ],
        out_specs=pl.BlockSpec((_TB,), lambda i: (i,)),
        out_shape=jax.ShapeDtypeStruct((_B,), jnp.int32),
    )(z2, e2, zb, cbtb)


_NW = 32
_BPW = _B // _NW
_CH = 128


def _gather_body(table_hbm, idx_hbm, out_hbm, idx_v, rows_v, sem):
    wid = lax.axis_index("s") * 2 + lax.axis_index("c")
    base = wid * _BPW
    pltpu.sync_copy(idx_hbm.at[pl.ds(base, _BPW)], idx_v)
    copies = [
        pltpu.async_copy(
            table_hbm.at[idx_v.at[pl.ds(j * _CH, _CH)]],
            rows_v.at[pl.ds(j * _CH, _CH)],
            sem,
        )
        for j in range(_BPW // _CH)
    ]
    for c in copies:
        c.wait()
    pltpu.sync_copy(rows_v, out_hbm.at[pl.ds(base, _BPW)])


@functools.cache
def _gather_rows():
    return pl.kernel(
        _gather_body,
        out_type=jax.ShapeDtypeStruct((_B, _D), jnp.float32),
        mesh=plsc.VectorSubcoreMesh(core_axis_name="c", subcore_axis_name="s"),
        scratch_types=[
            pltpu.VMEM((_BPW,), jnp.int32),
            pltpu.VMEM((_BPW, _D), jnp.float32),
            pltpu.SemaphoreType.DMA,
        ],
        compiler_params=pltpu.CompilerParams(use_tc_tiling_on_sc=False),
    )


def kernel(z_e, codebook):
    z2 = jnp.sum(z_e ** 2, axis=1, keepdims=True)
    e2 = jnp.sum(codebook ** 2, axis=1)[None, :]
    zb = z_e.astype(jnp.bfloat16)
    cbtb = codebook.T.astype(jnp.bfloat16)
    codes = _compute_codes(z2, e2, zb, cbtb)
    z_q = _gather_rows()(codebook, codes)
    return (z_q, codes)
